# baseline (speedup 1.0000x reference)
import jax
import jax.numpy as jnp
from jax import lax
import functools
from jax.experimental import pallas as pl
from jax.experimental.pallas import tpu as pltpu
from jax.experimental.pallas import tpu_sc as plsc

N_NODES = 10000
N_EDGES = 320000
D_EDGE = 16

NC = 2
NS = 16
NW = NC * NS

BATCH = 128
N_BATCHES = N_EDGES // BATCH
BATCHES_PER_TILE = N_BATCHES // NW
N_TAIL = N_BATCHES - BATCHES_PER_TILE * NW
BLOCKS_PER_TILE = 6
BATCHES_PER_BLOCK = BATCHES_PER_TILE // BLOCKS_PER_TILE
EDGES_PER_BLOCK = BATCHES_PER_BLOCK * BATCH

N_PAD = 10240
ACC_ROWS = N_PAD // NS
OUT_ROWS = 624
OUT_ROWS_LAST = N_NODES - (NS - 1) * OUT_ROWS

CMB_ROWS = 312
CMB_ROWS_LAST = N_NODES - (NW - 1) * CMB_ROWS

_MESH = plsc.VectorSubcoreMesh(core_axis_name="c", subcore_axis_name="s")
_SC_PARAMS = pltpu.CompilerParams(use_tc_tiling_on_sc=False)


def _sc_scatter(col1d, edge_attr, ones2d, zeros2d):
    @functools.partial(
        pl.kernel,
        mesh=_MESH,
        out_type=(
            jax.ShapeDtypeStruct((NC, N_NODES, D_EDGE), jnp.float32),
            jax.ShapeDtypeStruct((NC, N_NODES, D_EDGE), jnp.float32),
        ),
        scratch_types=[
            pltpu.VMEM((BATCHES_PER_TILE + 1, BATCH), jnp.int32),
            pltpu.VMEM((2, EDGES_PER_BLOCK, D_EDGE), jnp.float32),
            pltpu.VMEM((BATCH, D_EDGE), jnp.float32),
            pltpu.VMEM_SHARED((N_PAD, D_EDGE), jnp.float32),
            pltpu.VMEM_SHARED((N_PAD, D_EDGE), jnp.float32),
            pltpu.SemaphoreType.DMA,
            pltpu.SemaphoreType.DMA,
            pltpu.SemaphoreType.DMA,
        ],
        compiler_params=_SC_PARAMS,
    )
    def k(col_hbm, ea_hbm, ones_hbm, zeros_hbm, psums_hbm, pcnts_hbm,
          idx_v, rows_v, ones_v, sums_sh, cnts_sh, load_sem, idx_sem,
          scat_sem):
        c = lax.axis_index("c")
        s = lax.axis_index("s")
        w = c * NS + s
        b0 = w * BATCHES_PER_TILE

        idx_desc = [
            pltpu.async_copy(col_hbm.at[pl.ds((b0 + b) * BATCH, BATCH)],
                             idx_v.at[b], idx_sem)
            for b in range(BATCHES_PER_TILE)
        ]

        pltpu.sync_copy(zeros_hbm, sums_sh.at[pl.ds(s * ACC_ROWS, ACC_ROWS)])
        pltpu.sync_copy(zeros_hbm, cnts_sh.at[pl.ds(s * ACC_ROWS, ACC_ROWS)])
        pltpu.sync_copy(ones_hbm, ones_v)
        for d in idx_desc:
            d.wait()
        plsc.subcore_barrier()

        def start_rows_load(blk, buf):
            e0 = (b0 + blk * BATCHES_PER_BLOCK) * BATCH
            return pltpu.async_copy(ea_hbm.at[pl.ds(e0, EDGES_PER_BLOCK)],
                                    rows_v.at[buf], load_sem)

        pending = [[], []]
        load_desc = [None, None]
        load_desc[0] = start_rows_load(0, 0)
        for blk in range(BLOCKS_PER_TILE):
            cur = blk % 2
            nxt = 1 - cur
            load_desc[cur].wait()
            if blk + 1 < BLOCKS_PER_TILE:
                for d in pending[nxt]:
                    d.wait()
                pending[nxt] = []
                load_desc[nxt] = start_rows_load(blk + 1, nxt)
            for j in range(BATCHES_PER_BLOCK):
                bi = blk * BATCHES_PER_BLOCK + j
                pending[cur].append(pltpu.async_copy(
                    rows_v.at[cur, pl.ds(j * BATCH, BATCH)],
                    sums_sh.at[idx_v.at[bi]], scat_sem, add=True))
                pending[cur].append(pltpu.async_copy(
                    ones_v, cnts_sh.at[idx_v.at[bi]], scat_sem, add=True))
        for b in (0, 1):
            for d in pending[b]:
                d.wait()

        @pl.when(w < N_TAIL)
        def _():
            tb = NW * BATCHES_PER_TILE + w
            pltpu.sync_copy(col_hbm.at[pl.ds(tb * BATCH, BATCH)],
                            idx_v.at[BATCHES_PER_TILE])
            pltpu.sync_copy(ea_hbm.at[pl.ds(tb * BATCH, BATCH)],
                            rows_v.at[0, pl.ds(0, BATCH)])
            pltpu.sync_copy(rows_v.at[0, pl.ds(0, BATCH)],
                            sums_sh.at[idx_v.at[BATCHES_PER_TILE]], add=True)
            pltpu.sync_copy(ones_v, cnts_sh.at[idx_v.at[BATCHES_PER_TILE]],
                            add=True)

        plsc.subcore_barrier()

        def writeout(nrows):
            r0 = s * OUT_ROWS
            ds_ = [
                pltpu.async_copy(sums_sh.at[pl.ds(r0, nrows)],
                                 psums_hbm.at[c, pl.ds(r0, nrows)], load_sem),
                pltpu.async_copy(cnts_sh.at[pl.ds(r0, nrows)],
                                 pcnts_hbm.at[c, pl.ds(r0, nrows)], load_sem),
            ]
            for d in ds_:
                d.wait()

        @pl.when(s < NS - 1)
        def _():
            writeout(OUT_ROWS)

        @pl.when(s == NS - 1)
        def _():
            writeout(OUT_ROWS_LAST)

    return k(col1d, edge_attr, ones2d, zeros2d)


def _sc_combine(psums, pcnts):
    @functools.partial(
        pl.kernel,
        mesh=_MESH,
        out_type=jax.ShapeDtypeStruct((N_NODES, D_EDGE), jnp.float32),
        scratch_types=[
            pltpu.VMEM((CMB_ROWS_LAST, D_EDGE), jnp.float32),
            pltpu.VMEM((CMB_ROWS_LAST, D_EDGE), jnp.float32),
            pltpu.VMEM((CMB_ROWS_LAST, D_EDGE), jnp.float32),
            pltpu.VMEM((CMB_ROWS_LAST, D_EDGE), jnp.float32),
            pltpu.VMEM((CMB_ROWS_LAST, D_EDGE), jnp.float32),
            pltpu.SemaphoreType.DMA,
        ],
        compiler_params=_SC_PARAMS,
    )
    def k(ps_hbm, pc_hbm, out_hbm, s0_v, s1_v, c0_v, c1_v, o_v, sem):
        c = lax.axis_index("c")
        s = lax.axis_index("s")
        w = c * NS + s
        r0 = w * CMB_ROWS

        def run(nrows):
            ds_ = [
                pltpu.async_copy(ps_hbm.at[0, pl.ds(r0, nrows)],
                                 s0_v.at[pl.ds(0, nrows)], sem),
                pltpu.async_copy(ps_hbm.at[1, pl.ds(r0, nrows)],
                                 s1_v.at[pl.ds(0, nrows)], sem),
                pltpu.async_copy(pc_hbm.at[0, pl.ds(r0, nrows)],
                                 c0_v.at[pl.ds(0, nrows)], sem),
                pltpu.async_copy(pc_hbm.at[1, pl.ds(r0, nrows)],
                                 c1_v.at[pl.ds(0, nrows)], sem),
            ]
            for d in ds_:
                d.wait()

            def body(i, carry):
                sums = s0_v[i] + s1_v[i]
                cnts = c0_v[i] + c1_v[i]
                o_v[i] = sums / jnp.maximum(cnts, 1.0)
                return carry

            lax.fori_loop(0, nrows, body, 0)
            pltpu.sync_copy(o_v.at[pl.ds(0, nrows)],
                            out_hbm.at[pl.ds(r0, nrows)])

        @pl.when(w < NW - 1)
        def _():
            run(CMB_ROWS)

        @pl.when(w == NW - 1)
        def _():
            run(CMB_ROWS_LAST)

    return k(psums, pcnts)


def kernel(x, edge_index, edge_attr):
    col1d = edge_index[1].astype(jnp.int32)
    ones2d = jnp.ones((BATCH, D_EDGE), jnp.float32)
    zeros2d = jnp.zeros((ACC_ROWS, D_EDGE), jnp.float32)

    psums, pcnts = _sc_scatter(col1d, edge_attr, ones2d, zeros2d)
    return _sc_combine(psums, pcnts)


# speedup vs baseline: 1.0005x; 1.0005x over previous
import jax
import jax.numpy as jnp
from jax import lax
import functools
from jax.experimental import pallas as pl
from jax.experimental.pallas import tpu as pltpu
from jax.experimental.pallas import tpu_sc as plsc

N_NODES = 10000
N_EDGES = 320000
D_EDGE = 16

NC = 2
NS = 16
NW = NC * NS

BATCH = 128
N_BATCHES = N_EDGES // BATCH
BATCHES_PER_TILE = N_BATCHES // NW
N_TAIL = N_BATCHES - BATCHES_PER_TILE * NW
BLOCKS_PER_TILE = 6
BATCHES_PER_BLOCK = BATCHES_PER_TILE // BLOCKS_PER_TILE
EDGES_PER_BLOCK = BATCHES_PER_BLOCK * BATCH

N_PAD = 10240
ACC_ROWS = N_PAD // NS
OUT_ROWS = 624
OUT_ROWS_LAST = N_NODES - (NS - 1) * OUT_ROWS

CMB_ROWS = 312
CMB_ROWS_LAST = N_NODES - (NW - 1) * CMB_ROWS

_MESH = plsc.VectorSubcoreMesh(core_axis_name="c", subcore_axis_name="s")
_SC_PARAMS = pltpu.CompilerParams(use_tc_tiling_on_sc=False)


def _sc_scatter(col2d, edge_attr, ones2d, zeros2d):
    @functools.partial(
        pl.kernel,
        mesh=_MESH,
        out_type=(
            jax.ShapeDtypeStruct((NC, N_NODES, D_EDGE), jnp.float32),
            jax.ShapeDtypeStruct((NC, N_NODES, D_EDGE), jnp.float32),
        ),
        scratch_types=[
            pltpu.VMEM((BATCHES_PER_TILE + 1, BATCH), jnp.int32),
            pltpu.VMEM((2, EDGES_PER_BLOCK, D_EDGE), jnp.float32),
            pltpu.VMEM((BATCH, D_EDGE), jnp.float32),
            pltpu.VMEM_SHARED((N_PAD, D_EDGE), jnp.float32),
            pltpu.VMEM_SHARED((N_PAD, D_EDGE), jnp.float32),
            pltpu.SemaphoreType.DMA,
            pltpu.SemaphoreType.DMA,
            pltpu.SemaphoreType.DMA,
        ],
        compiler_params=_SC_PARAMS,
    )
    def k(col_hbm, ea_hbm, ones_hbm, zeros_hbm, psums_hbm, pcnts_hbm,
          idx_v, rows_v, ones_v, sums_sh, cnts_sh, load_sem, idx_sem,
          scat_sem):
        c = lax.axis_index("c")
        s = lax.axis_index("s")
        w = c * NS + s
        b0 = w * BATCHES_PER_TILE

        idx_desc = [
            pltpu.async_copy(col_hbm.at[N_BATCHES + b0 + b],
                             idx_v.at[b], idx_sem)
            for b in range(BATCHES_PER_TILE)
        ]

        pltpu.sync_copy(zeros_hbm, sums_sh.at[pl.ds(s * ACC_ROWS, ACC_ROWS)])
        pltpu.sync_copy(zeros_hbm, cnts_sh.at[pl.ds(s * ACC_ROWS, ACC_ROWS)])
        pltpu.sync_copy(ones_hbm, ones_v)
        for d in idx_desc:
            d.wait()
        plsc.subcore_barrier()

        def start_rows_load(blk, buf):
            e0 = (b0 + blk * BATCHES_PER_BLOCK) * BATCH
            return pltpu.async_copy(ea_hbm.at[pl.ds(e0, EDGES_PER_BLOCK)],
                                    rows_v.at[buf], load_sem)

        pending = [[], []]
        load_desc = [None, None]
        load_desc[0] = start_rows_load(0, 0)
        for blk in range(BLOCKS_PER_TILE):
            cur = blk % 2
            nxt = 1 - cur
            load_desc[cur].wait()
            if blk + 1 < BLOCKS_PER_TILE:
                for d in pending[nxt]:
                    d.wait()
                pending[nxt] = []
                load_desc[nxt] = start_rows_load(blk + 1, nxt)
            for j in range(BATCHES_PER_BLOCK):
                bi = blk * BATCHES_PER_BLOCK + j
                pending[cur].append(pltpu.async_copy(
                    rows_v.at[cur, pl.ds(j * BATCH, BATCH)],
                    sums_sh.at[idx_v.at[bi]], scat_sem, add=True))
                pending[cur].append(pltpu.async_copy(
                    ones_v, cnts_sh.at[idx_v.at[bi]], scat_sem, add=True))
        for b in (0, 1):
            for d in pending[b]:
                d.wait()

        @pl.when(w < N_TAIL)
        def _():
            tb = NW * BATCHES_PER_TILE + w
            pltpu.sync_copy(col_hbm.at[N_BATCHES + tb],
                            idx_v.at[BATCHES_PER_TILE])
            pltpu.sync_copy(ea_hbm.at[pl.ds(tb * BATCH, BATCH)],
                            rows_v.at[0, pl.ds(0, BATCH)])
            pltpu.sync_copy(rows_v.at[0, pl.ds(0, BATCH)],
                            sums_sh.at[idx_v.at[BATCHES_PER_TILE]], add=True)
            pltpu.sync_copy(ones_v, cnts_sh.at[idx_v.at[BATCHES_PER_TILE]],
                            add=True)

        plsc.subcore_barrier()

        def writeout(nrows):
            r0 = s * OUT_ROWS
            ds_ = [
                pltpu.async_copy(sums_sh.at[pl.ds(r0, nrows)],
                                 psums_hbm.at[c, pl.ds(r0, nrows)], load_sem),
                pltpu.async_copy(cnts_sh.at[pl.ds(r0, nrows)],
                                 pcnts_hbm.at[c, pl.ds(r0, nrows)], load_sem),
            ]
            for d in ds_:
                d.wait()

        @pl.when(s < NS - 1)
        def _():
            writeout(OUT_ROWS)

        @pl.when(s == NS - 1)
        def _():
            writeout(OUT_ROWS_LAST)

    return k(col2d, edge_attr, ones2d, zeros2d)


def _sc_combine(psums, pcnts):
    @functools.partial(
        pl.kernel,
        mesh=_MESH,
        out_type=jax.ShapeDtypeStruct((N_NODES, D_EDGE), jnp.float32),
        scratch_types=[
            pltpu.VMEM((CMB_ROWS_LAST, D_EDGE), jnp.float32),
            pltpu.VMEM((CMB_ROWS_LAST, D_EDGE), jnp.float32),
            pltpu.VMEM((CMB_ROWS_LAST, D_EDGE), jnp.float32),
            pltpu.VMEM((CMB_ROWS_LAST, D_EDGE), jnp.float32),
            pltpu.VMEM((CMB_ROWS_LAST, D_EDGE), jnp.float32),
            pltpu.SemaphoreType.DMA,
        ],
        compiler_params=_SC_PARAMS,
    )
    def k(ps_hbm, pc_hbm, out_hbm, s0_v, s1_v, c0_v, c1_v, o_v, sem):
        c = lax.axis_index("c")
        s = lax.axis_index("s")
        w = c * NS + s
        r0 = w * CMB_ROWS

        def run(nrows):
            ds_ = [
                pltpu.async_copy(ps_hbm.at[0, pl.ds(r0, nrows)],
                                 s0_v.at[pl.ds(0, nrows)], sem),
                pltpu.async_copy(ps_hbm.at[1, pl.ds(r0, nrows)],
                                 s1_v.at[pl.ds(0, nrows)], sem),
                pltpu.async_copy(pc_hbm.at[0, pl.ds(r0, nrows)],
                                 c0_v.at[pl.ds(0, nrows)], sem),
                pltpu.async_copy(pc_hbm.at[1, pl.ds(r0, nrows)],
                                 c1_v.at[pl.ds(0, nrows)], sem),
            ]
            for d in ds_:
                d.wait()

            def body(i, carry):
                sums = s0_v[i] + s1_v[i]
                cnts = c0_v[i] + c1_v[i]
                o_v[i] = sums / jnp.maximum(cnts, 1.0)
                return carry

            lax.fori_loop(0, nrows, body, 0)
            pltpu.sync_copy(o_v.at[pl.ds(0, nrows)],
                            out_hbm.at[pl.ds(r0, nrows)])

        @pl.when(w < NW - 1)
        def _():
            run(CMB_ROWS)

        @pl.when(w == NW - 1)
        def _():
            run(CMB_ROWS_LAST)

    return k(psums, pcnts)


def kernel(x, edge_index, edge_attr):
    col2d = edge_index.astype(jnp.int32).reshape(2 * N_BATCHES, BATCH)
    ones2d = jnp.ones((BATCH, D_EDGE), jnp.float32)
    zeros2d = jnp.zeros((ACC_ROWS, D_EDGE), jnp.float32)

    psums, pcnts = _sc_scatter(col2d, edge_attr, ones2d, zeros2d)
    return _sc_combine(psums, pcnts)


# speedup vs baseline: 1.0021x; 1.0016x over previous
import jax
import jax.numpy as jnp
from jax import lax
import functools
from jax.experimental import pallas as pl
from jax.experimental.pallas import tpu as pltpu
from jax.experimental.pallas import tpu_sc as plsc

N_NODES = 10000
N_EDGES = 320000
D_EDGE = 16

NC = 2
NS = 16
NW = NC * NS

BATCH = 128
N_BATCHES = N_EDGES // BATCH
BATCHES_PER_TILE = N_BATCHES // NW
N_TAIL = N_BATCHES - BATCHES_PER_TILE * NW
BLOCKS_PER_TILE = 6
BATCHES_PER_BLOCK = BATCHES_PER_TILE // BLOCKS_PER_TILE
EDGES_PER_BLOCK = BATCHES_PER_BLOCK * BATCH

N_PAD = 10240
ACC_ROWS = N_PAD // NS
OUT_ROWS = 624
OUT_ROWS_LAST = N_NODES - (NS - 1) * OUT_ROWS

CMB_ROWS = 312
CMB_ROWS_LAST = N_NODES - (NW - 1) * CMB_ROWS

_MESH = plsc.VectorSubcoreMesh(core_axis_name="c", subcore_axis_name="s")
_SC_PARAMS = pltpu.CompilerParams(use_tc_tiling_on_sc=False)


def _sc_scatter(col3d, edge_attr, ones2d, zeros2d):
    @functools.partial(
        pl.kernel,
        mesh=_MESH,
        out_type=(
            jax.ShapeDtypeStruct((NC, N_NODES, D_EDGE), jnp.float32),
            jax.ShapeDtypeStruct((NC, N_NODES, D_EDGE), jnp.float32),
        ),
        scratch_types=[
            pltpu.VMEM((BATCHES_PER_TILE + 1, BATCH), jnp.int32),
            pltpu.VMEM((2, EDGES_PER_BLOCK, D_EDGE), jnp.float32),
            pltpu.VMEM((BATCH, D_EDGE), jnp.float32),
            pltpu.VMEM_SHARED((N_PAD, D_EDGE), jnp.float32),
            pltpu.VMEM_SHARED((N_PAD, D_EDGE), jnp.float32),
            pltpu.SemaphoreType.DMA,
            pltpu.SemaphoreType.DMA,
            pltpu.SemaphoreType.DMA,
        ],
        compiler_params=_SC_PARAMS,
    )
    def k(col_hbm, ea_hbm, ones_hbm, zeros_hbm, psums_hbm, pcnts_hbm,
          idx_v, rows_v, ones_v, sums_sh, cnts_sh, load_sem, idx_sem,
          scat_sem):
        c = lax.axis_index("c")
        s = lax.axis_index("s")
        w = c * NS + s
        b0 = w * BATCHES_PER_TILE

        def col_row(g):
            return col_hbm.at[g // 4, 2 * (g % 4) + 1]

        idx_desc = [
            pltpu.async_copy(col_row(b0 + b), idx_v.at[b], idx_sem)
            for b in range(BATCHES_PER_TILE)
        ]

        pltpu.sync_copy(zeros_hbm, sums_sh.at[pl.ds(s * ACC_ROWS, ACC_ROWS)])
        pltpu.sync_copy(zeros_hbm, cnts_sh.at[pl.ds(s * ACC_ROWS, ACC_ROWS)])
        pltpu.sync_copy(ones_hbm, ones_v)
        for d in idx_desc:
            d.wait()
        plsc.subcore_barrier()

        def start_rows_load(blk, buf):
            e0 = (b0 + blk * BATCHES_PER_BLOCK) * BATCH
            return pltpu.async_copy(ea_hbm.at[pl.ds(e0, EDGES_PER_BLOCK)],
                                    rows_v.at[buf], load_sem)

        pending = [[], []]
        load_desc = [None, None]
        load_desc[0] = start_rows_load(0, 0)
        for blk in range(BLOCKS_PER_TILE):
            cur = blk % 2
            nxt = 1 - cur
            load_desc[cur].wait()
            if blk + 1 < BLOCKS_PER_TILE:
                for d in pending[nxt]:
                    d.wait()
                pending[nxt] = []
                load_desc[nxt] = start_rows_load(blk + 1, nxt)
            for j in range(BATCHES_PER_BLOCK):
                bi = blk * BATCHES_PER_BLOCK + j
                pending[cur].append(pltpu.async_copy(
                    rows_v.at[cur, pl.ds(j * BATCH, BATCH)],
                    sums_sh.at[idx_v.at[bi]], scat_sem, add=True))
                pending[cur].append(pltpu.async_copy(
                    ones_v, cnts_sh.at[idx_v.at[bi]], scat_sem, add=True))
        for b in (0, 1):
            for d in pending[b]:
                d.wait()

        @pl.when(w < N_TAIL)
        def _():
            tb = NW * BATCHES_PER_TILE + w
            pltpu.sync_copy(col_row(tb), idx_v.at[BATCHES_PER_TILE])
            pltpu.sync_copy(ea_hbm.at[pl.ds(tb * BATCH, BATCH)],
                            rows_v.at[0, pl.ds(0, BATCH)])
            pltpu.sync_copy(rows_v.at[0, pl.ds(0, BATCH)],
                            sums_sh.at[idx_v.at[BATCHES_PER_TILE]], add=True)
            pltpu.sync_copy(ones_v, cnts_sh.at[idx_v.at[BATCHES_PER_TILE]],
                            add=True)

        plsc.subcore_barrier()

        def writeout(nrows):
            r0 = s * OUT_ROWS
            ds_ = [
                pltpu.async_copy(sums_sh.at[pl.ds(r0, nrows)],
                                 psums_hbm.at[c, pl.ds(r0, nrows)], load_sem),
                pltpu.async_copy(cnts_sh.at[pl.ds(r0, nrows)],
                                 pcnts_hbm.at[c, pl.ds(r0, nrows)], load_sem),
            ]
            for d in ds_:
                d.wait()

        @pl.when(s < NS - 1)
        def _():
            writeout(OUT_ROWS)

        @pl.when(s == NS - 1)
        def _():
            writeout(OUT_ROWS_LAST)

    return k(col3d, edge_attr, ones2d, zeros2d)


def _sc_combine(psums, pcnts):
    @functools.partial(
        pl.kernel,
        mesh=_MESH,
        out_type=jax.ShapeDtypeStruct((N_NODES, D_EDGE), jnp.float32),
        scratch_types=[
            pltpu.VMEM((CMB_ROWS_LAST, D_EDGE), jnp.float32),
            pltpu.VMEM((CMB_ROWS_LAST, D_EDGE), jnp.float32),
            pltpu.VMEM((CMB_ROWS_LAST, D_EDGE), jnp.float32),
            pltpu.VMEM((CMB_ROWS_LAST, D_EDGE), jnp.float32),
            pltpu.VMEM((CMB_ROWS_LAST, D_EDGE), jnp.float32),
            pltpu.SemaphoreType.DMA,
        ],
        compiler_params=_SC_PARAMS,
    )
    def k(ps_hbm, pc_hbm, out_hbm, s0_v, s1_v, c0_v, c1_v, o_v, sem):
        c = lax.axis_index("c")
        s = lax.axis_index("s")
        w = c * NS + s
        r0 = w * CMB_ROWS

        def run(nrows):
            ds_ = [
                pltpu.async_copy(ps_hbm.at[0, pl.ds(r0, nrows)],
                                 s0_v.at[pl.ds(0, nrows)], sem),
                pltpu.async_copy(ps_hbm.at[1, pl.ds(r0, nrows)],
                                 s1_v.at[pl.ds(0, nrows)], sem),
                pltpu.async_copy(pc_hbm.at[0, pl.ds(r0, nrows)],
                                 c0_v.at[pl.ds(0, nrows)], sem),
                pltpu.async_copy(pc_hbm.at[1, pl.ds(r0, nrows)],
                                 c1_v.at[pl.ds(0, nrows)], sem),
            ]
            for d in ds_:
                d.wait()

            def body(i, carry):
                sums = s0_v[i] + s1_v[i]
                cnts = c0_v[i] + c1_v[i]
                o_v[i] = sums / jnp.maximum(cnts, 1.0)
                return carry

            lax.fori_loop(0, nrows, body, 0)
            pltpu.sync_copy(o_v.at[pl.ds(0, nrows)],
                            out_hbm.at[pl.ds(r0, nrows)])

        @pl.when(w < NW - 1)
        def _():
            run(CMB_ROWS)

        @pl.when(w == NW - 1)
        def _():
            run(CMB_ROWS_LAST)

    return k(psums, pcnts)


def kernel(x, edge_index, edge_attr):
    col3d = (edge_index.astype(jnp.int32)
             .reshape(2, N_BATCHES, BATCH)
             .transpose(1, 0, 2)
             .reshape(N_BATCHES // 4, 8, BATCH))
    ones2d = jnp.ones((BATCH, D_EDGE), jnp.float32)
    zeros2d = jnp.zeros((ACC_ROWS, D_EDGE), jnp.float32)

    psums, pcnts = _sc_scatter(col3d, edge_attr, ones2d, zeros2d)
    return _sc_combine(psums, pcnts)
